# baseline (speedup 1.0000x reference)
import functools

import jax
import jax.numpy as jnp
from jax import lax
from jax.experimental import pallas as pl
from jax.experimental.pallas import tpu as pltpu
from jax.experimental.pallas import tpu_sc as plsc

N_NODES = 10000
D_FEAT = 128
N_EDGES = 320000

NC = 2
NS = 16
L = 16
NW = NC * NS
E_W = N_EDGES // NW
B = 80
CH = E_W // B
G = B // L
P = D_FEAT // 2
K = 8


def _tile_body(xu_hbm, xm_hbm, uidx_hbm, midx_hbm, out_hbm,
               uidx_v, midx_v, u0, m0, u1, m1, out_v, sem0, sem1):
    wid = lax.axis_index("s") * NC + lax.axis_index("c")
    base = wid * E_W

    pltpu.sync_copy(uidx_hbm.at[pl.ds(base, E_W)], uidx_v)
    pltpu.sync_copy(midx_hbm.at[pl.ds(base, E_W)], midx_v)

    bufs = ((u0, m0, sem0), (u1, m1, sem1))

    def start(c, b):
        ub, mb, sem = bufs[b]
        pltpu.async_copy(xu_hbm.at[uidx_v.at[pl.ds(c * B, B)]], ub, sem)
        pltpu.async_copy(xm_hbm.at[midx_v.at[pl.ds(c * B, B)]], mb, sem)

    def drain(b):
        ub, mb, sem = bufs[b]
        pltpu.make_async_copy(xu_hbm.at[uidx_v.at[pl.ds(0, B)]], ub,
                              sem).wait()
        pltpu.make_async_copy(xm_hbm.at[uidx_v.at[pl.ds(0, B)]], mb,
                              sem).wait()

    def compute(c, b):
        ub, mb, _ = bufs[b]
        off = c * B
        for g in range(G):
            rows = jnp.arange(L, dtype=jnp.int32) + g * L
            zero = jnp.zeros((L,), jnp.float32)
            cols0 = jnp.arange(L, dtype=jnp.int32)

            def p_body(_, carry):
                cols, *accs = carry
                new_accs = []
                for k in range(K):
                    col = ((cols + k) if k else cols) & (P - 1)
                    u32 = plsc.load_gather(ub, [rows, col])
                    m32 = plsc.load_gather(mb, [rows, col])
                    ulo = plsc.bitcast(u32 << 16, jnp.float32)
                    mlo = plsc.bitcast(m32 << 16, jnp.float32)
                    uhi = plsc.bitcast(u32, jnp.float32)
                    mhi = plsc.bitcast(m32, jnp.float32)
                    lo = accs[2 * k] + ulo * mlo
                    hi = accs[2 * k + 1] + uhi * mhi
                    new_accs += [lo, hi]
                return (cols + K, *new_accs)

            res = lax.fori_loop(0, P // K, p_body,
                                (cols0,) + (zero,) * (2 * K))
            accs = list(res[1:])
            while len(accs) > 1:
                accs = [a + b_ for a, b_ in zip(accs[::2], accs[1::2])]
            out_v[pl.ds(off + g * L, L)] = accs[0]

    start(0, 0)

    def pair_body(j, carry):
        c0 = 2 * j
        start(c0 + 1, 1)
        drain(0)
        compute(c0, 0)
        start(c0 + 2, 0)
        drain(1)
        compute(c0 + 1, 1)
        return carry

    lax.fori_loop(0, (CH - 1) // 2, pair_body, 0)
    drain(0)
    compute(CH - 1, 0)

    pltpu.sync_copy(out_v, out_hbm.at[pl.ds(base, E_W)])


@functools.partial(
    pl.kernel,
    mesh=plsc.VectorSubcoreMesh(core_axis_name="c", subcore_axis_name="s"),
    out_type=jax.ShapeDtypeStruct((N_EDGES,), jnp.float32),
    compiler_params=pltpu.CompilerParams(needs_layout_passes=False,
                                         use_tc_tiling_on_sc=False),
    scratch_types=[
        pltpu.VMEM((E_W,), jnp.int32),
        pltpu.VMEM((E_W,), jnp.int32),
        pltpu.VMEM((B, P), jnp.int32),
        pltpu.VMEM((B, P), jnp.int32),
        pltpu.VMEM((B, P), jnp.int32),
        pltpu.VMEM((B, P), jnp.int32),
        pltpu.VMEM((E_W,), jnp.float32),
        pltpu.SemaphoreType.DMA,
        pltpu.SemaphoreType.DMA,
    ],
)
def _edge_dot_sc(xu_hbm, xm_hbm, uidx_hbm, midx_hbm, out_hbm,
                 uidx_v, midx_v, u0, m0, u1, m1, out_v, sem0, sem1):
    _tile_body(xu_hbm, xm_hbm, uidx_hbm, midx_hbm, out_hbm,
               uidx_v, midx_v, u0, m0, u1, m1, out_v, sem0, sem1)


def kernel(x_user, x_movie, edge_label_index):
    idx = edge_label_index.astype(jnp.int32)
    xu_p = lax.bitcast_convert_type(
        x_user.astype(jnp.bfloat16).reshape(N_NODES, P, 2), jnp.int32)
    xm_p = lax.bitcast_convert_type(
        x_movie.astype(jnp.bfloat16).reshape(N_NODES, P, 2), jnp.int32)
    return _edge_dot_sc(xu_p, xm_p, idx[0], idx[1])


# speedup vs baseline: 1.3020x; 1.3020x over previous
import functools

import jax
import jax.numpy as jnp
from jax import lax
from jax.experimental import pallas as pl
from jax.experimental.pallas import tpu as pltpu
from jax.experimental.pallas import tpu_sc as plsc

N_NODES = 10000
D_FEAT = 128
N_EDGES = 320000

NC = 2
NS = 16
L = 16
NW = NC * NS
E_W = N_EDGES // NW
B = 80
CH = E_W // B
G = B // L
K = 8
NBUF = 4


def _tile_body(xu_hbm, xm_hbm, uidx_hbm, midx_hbm, out_hbm,
               uidx_v, midx_v, u0, m0, u1, m1, u2, m2, u3, m3, out_v,
               sem0, sem1, sem2, sem3):
    wid = lax.axis_index("s") * NC + lax.axis_index("c")
    base = wid * E_W

    pltpu.sync_copy(uidx_hbm.at[pl.ds(base, E_W)], uidx_v)
    pltpu.sync_copy(midx_hbm.at[pl.ds(base, E_W)], midx_v)

    bufs = ((u0, m0, sem0), (u1, m1, sem1), (u2, m2, sem2), (u3, m3, sem3))

    def start(c, b):
        ub, mb, sem = bufs[b]
        pltpu.async_copy(xu_hbm.at[uidx_v.at[pl.ds(c * B, B)]], ub, sem)
        pltpu.async_copy(xm_hbm.at[midx_v.at[pl.ds(c * B, B)]], mb, sem)

    def drain(b):
        ub, mb, sem = bufs[b]
        pltpu.make_async_copy(xu_hbm.at[uidx_v.at[pl.ds(0, B)]], ub,
                              sem).wait()
        pltpu.make_async_copy(xm_hbm.at[uidx_v.at[pl.ds(0, B)]], mb,
                              sem).wait()

    def compute(c, b):
        ub, mb, _ = bufs[b]
        off = c * B
        for g in range(G):
            rows = jnp.arange(L, dtype=jnp.int32) + g * L
            zero = jnp.zeros((L,), jnp.float32)
            cols0 = jnp.arange(L, dtype=jnp.int32)

            def d_body(_, carry):
                cols, *accs = carry
                new_accs = []
                for k in range(K):
                    col = ((cols + k) if k else cols) & (D_FEAT - 1)
                    uv = plsc.load_gather(ub, [rows, col])
                    mv = plsc.load_gather(mb, [rows, col])
                    new_accs.append(accs[k] + uv * mv)
                return (cols + K, *new_accs)

            res = lax.fori_loop(0, D_FEAT // K, d_body,
                                (cols0,) + (zero,) * K)
            accs = list(res[1:])
            while len(accs) > 1:
                accs = [a + b_ for a, b_ in zip(accs[::2], accs[1::2])]
            out_v[pl.ds(off + g * L, L)] = accs[0]

    for b in range(NBUF):
        start(b, b)

    def quad_body(j, carry):
        c0 = NBUF * j
        for b in range(NBUF):
            drain(b)
            compute(c0 + b, b)
            start(c0 + b + NBUF, b)
        return carry

    lax.fori_loop(0, (CH - (NBUF + 1)) // NBUF, quad_body, 0)

    c0 = ((CH - (NBUF + 1)) // NBUF) * NBUF
    drain(0)
    compute(c0, 0)
    start(CH - 1, 0)
    for b in range(1, NBUF):
        drain(b)
        compute(c0 + b, b)
    drain(0)
    compute(CH - 1, 0)

    pltpu.sync_copy(out_v, out_hbm.at[pl.ds(base, E_W)])


@functools.partial(
    pl.kernel,
    mesh=plsc.VectorSubcoreMesh(core_axis_name="c", subcore_axis_name="s"),
    out_type=jax.ShapeDtypeStruct((N_EDGES,), jnp.float32),
    compiler_params=pltpu.CompilerParams(needs_layout_passes=False),
    scratch_types=[
        pltpu.VMEM((E_W,), jnp.int32),
        pltpu.VMEM((E_W,), jnp.int32),
        pltpu.VMEM((B, D_FEAT), jnp.float32),
        pltpu.VMEM((B, D_FEAT), jnp.float32),
        pltpu.VMEM((B, D_FEAT), jnp.float32),
        pltpu.VMEM((B, D_FEAT), jnp.float32),
        pltpu.VMEM((B, D_FEAT), jnp.float32),
        pltpu.VMEM((B, D_FEAT), jnp.float32),
        pltpu.VMEM((B, D_FEAT), jnp.float32),
        pltpu.VMEM((B, D_FEAT), jnp.float32),
        pltpu.VMEM((E_W,), jnp.float32),
        pltpu.SemaphoreType.DMA,
        pltpu.SemaphoreType.DMA,
        pltpu.SemaphoreType.DMA,
        pltpu.SemaphoreType.DMA,
    ],
)
def _edge_dot_sc(xu_hbm, xm_hbm, uidx_hbm, midx_hbm, out_hbm,
                 uidx_v, midx_v, u0, m0, u1, m1, u2, m2, u3, m3, out_v,
                 sem0, sem1, sem2, sem3):
    _tile_body(xu_hbm, xm_hbm, uidx_hbm, midx_hbm, out_hbm,
               uidx_v, midx_v, u0, m0, u1, m1, u2, m2, u3, m3, out_v,
               sem0, sem1, sem2, sem3)


def kernel(x_user, x_movie, edge_label_index):
    idx = edge_label_index.astype(jnp.int32)
    return _edge_dot_sc(x_user, x_movie, idx[0], idx[1])


# speedup vs baseline: 1.3022x; 1.0001x over previous
import functools

import jax
import jax.numpy as jnp
from jax import lax
from jax.experimental import pallas as pl
from jax.experimental.pallas import tpu as pltpu
from jax.experimental.pallas import tpu_sc as plsc

N_NODES = 10000
D_FEAT = 128
N_EDGES = 320000

NC = 2
NS = 16
L = 16
NW = NC * NS
E_W = N_EDGES // NW
B = 80
CH = E_W // B
G = B // L
K = 8
NBUF = 5


def _tile_body(xu_hbm, xm_hbm, uidx_hbm, midx_hbm, out_hbm,
               uidx_v, midx_v, u0, m0, u1, m1, u2, m2, u3, m3, u4, m4,
               ob0, ob1, ob2, ob3, ob4,
               sem0, sem1, sem2, sem3, sem4,
               semo0, semo1, semo2, semo3, semo4):
    wid = lax.axis_index("s") * NC + lax.axis_index("c")
    base = wid * E_W

    pltpu.sync_copy(uidx_hbm.at[pl.ds(base, E_W)], uidx_v)
    pltpu.sync_copy(midx_hbm.at[pl.ds(base, E_W)], midx_v)

    bufs = ((u0, m0, sem0), (u1, m1, sem1), (u2, m2, sem2), (u3, m3, sem3),
            (u4, m4, sem4))
    obufs = ((ob0, semo0), (ob1, semo1), (ob2, semo2), (ob3, semo3),
             (ob4, semo4))

    def start_out(c, b):
        ob, sem = obufs[b]
        pltpu.async_copy(ob, out_hbm.at[pl.ds(base + c * B, B)], sem)

    def wait_out(b):
        ob, sem = obufs[b]
        pltpu.make_async_copy(ob, out_hbm.at[pl.ds(base, B)], sem).wait()

    def start(c, b):
        ub, mb, sem = bufs[b]
        pltpu.async_copy(xu_hbm.at[uidx_v.at[pl.ds(c * B, B)]], ub, sem)
        pltpu.async_copy(xm_hbm.at[midx_v.at[pl.ds(c * B, B)]], mb, sem)

    def drain(b):
        ub, mb, sem = bufs[b]
        pltpu.make_async_copy(xu_hbm.at[uidx_v.at[pl.ds(0, B)]], ub,
                              sem).wait()
        pltpu.make_async_copy(xm_hbm.at[uidx_v.at[pl.ds(0, B)]], mb,
                              sem).wait()

    def compute(b):
        ub, mb, _ = bufs[b]
        ob = obufs[b][0]
        for g in range(G):
            rows = jnp.arange(L, dtype=jnp.int32) + g * L
            zero = jnp.zeros((L,), jnp.float32)
            cols0 = jnp.arange(L, dtype=jnp.int32)

            def d_body(_, carry):
                cols, *accs = carry
                new_accs = []
                for k in range(K):
                    col = ((cols + k) if k else cols) & (D_FEAT - 1)
                    uv = plsc.load_gather(ub, [rows, col])
                    mv = plsc.load_gather(mb, [rows, col])
                    new_accs.append(accs[k] + uv * mv)
                return (cols + K, *new_accs)

            res = lax.fori_loop(0, D_FEAT // K, d_body,
                                (cols0,) + (zero,) * K)
            accs = list(res[1:])
            while len(accs) > 1:
                accs = [a + b_ for a, b_ in zip(accs[::2], accs[1::2])]
            ob[pl.ds(g * L, L)] = accs[0]

    for b in range(NBUF):
        start(b, b)

    def quint_body(j, carry):
        c0 = NBUF * j
        for b in range(NBUF):
            c = c0 + b
            drain(b)

            @pl.when(c >= NBUF)
            def _():
                wait_out(b)

            compute(b)
            start_out(c, b)

            @pl.when(c + NBUF <= CH - 1)
            def _():
                start(c + NBUF, b)
        return carry

    lax.fori_loop(0, CH // NBUF, quint_body, 0)
    for b in range(NBUF):
        wait_out(b)


@functools.partial(
    pl.kernel,
    mesh=plsc.VectorSubcoreMesh(core_axis_name="c", subcore_axis_name="s"),
    out_type=jax.ShapeDtypeStruct((N_EDGES,), jnp.float32),
    compiler_params=pltpu.CompilerParams(needs_layout_passes=False),
    scratch_types=[
        pltpu.VMEM((E_W,), jnp.int32),
        pltpu.VMEM((E_W,), jnp.int32),
        pltpu.VMEM((B, D_FEAT), jnp.float32),
        pltpu.VMEM((B, D_FEAT), jnp.float32),
        pltpu.VMEM((B, D_FEAT), jnp.float32),
        pltpu.VMEM((B, D_FEAT), jnp.float32),
        pltpu.VMEM((B, D_FEAT), jnp.float32),
        pltpu.VMEM((B, D_FEAT), jnp.float32),
        pltpu.VMEM((B, D_FEAT), jnp.float32),
        pltpu.VMEM((B, D_FEAT), jnp.float32),
        pltpu.VMEM((B, D_FEAT), jnp.float32),
        pltpu.VMEM((B, D_FEAT), jnp.float32),
        pltpu.VMEM((B,), jnp.float32),
        pltpu.VMEM((B,), jnp.float32),
        pltpu.VMEM((B,), jnp.float32),
        pltpu.VMEM((B,), jnp.float32),
        pltpu.VMEM((B,), jnp.float32),
        pltpu.SemaphoreType.DMA,
        pltpu.SemaphoreType.DMA,
        pltpu.SemaphoreType.DMA,
        pltpu.SemaphoreType.DMA,
        pltpu.SemaphoreType.DMA,
        pltpu.SemaphoreType.DMA,
        pltpu.SemaphoreType.DMA,
        pltpu.SemaphoreType.DMA,
        pltpu.SemaphoreType.DMA,
        pltpu.SemaphoreType.DMA,
    ],
)
def _edge_dot_sc(xu_hbm, xm_hbm, uidx_hbm, midx_hbm, out_hbm,
                 uidx_v, midx_v, u0, m0, u1, m1, u2, m2, u3, m3, u4, m4,
                 ob0, ob1, ob2, ob3, ob4,
                 sem0, sem1, sem2, sem3, sem4,
                 semo0, semo1, semo2, semo3, semo4):
    _tile_body(xu_hbm, xm_hbm, uidx_hbm, midx_hbm, out_hbm,
               uidx_v, midx_v, u0, m0, u1, m1, u2, m2, u3, m3, u4, m4,
               ob0, ob1, ob2, ob3, ob4,
               sem0, sem1, sem2, sem3, sem4,
               semo0, semo1, semo2, semo3, semo4)


def kernel(x_user, x_movie, edge_label_index):
    idx = edge_label_index.astype(jnp.int32)
    return _edge_dot_sc(x_user, x_movie, idx[0], idx[1])
